# baseline (speedup 1.0000x reference)
import functools

import jax
import jax.numpy as jnp
from jax import lax
from jax.experimental import pallas as pl
from jax.experimental.pallas import tpu as pltpu
from jax.experimental.pallas import tpu_sc as plsc

B = 16384
HIDDEN = 256
ACTIONS = 64
EMB = 16
NFIELDS = 17
CONCAT = NFIELDS * EMB
TABLE_ROWS = 25 + (NFIELDS - 1) * 10

GPB = 24
PADW = GPB * EMB
NC, NS = 2, 16
NW = NC * NS
R = B * GPB
R_PER_W = R // NW
IDX_MINOR = 128
B_PER_CHUNK = 128
CHUNK = B_PER_CHUNK * GPB
STREAMS_PER_CHUNK = CHUNK // IDX_MINOR
B_PER_W = B // NW
NCHUNK = B_PER_W // B_PER_CHUNK


def _sc_gather(flat_table, gidx_flat):
    mesh = plsc.VectorSubcoreMesh(core_axis_name="c", subcore_axis_name="s")

    @functools.partial(
        pl.kernel,
        mesh=mesh,
        out_type=jax.ShapeDtypeStruct((R, EMB), jnp.float32),
        scratch_types=[
            pltpu.VMEM((TABLE_ROWS, EMB), jnp.float32),
            pltpu.VMEM_SHARED((TABLE_ROWS, EMB), jnp.float32),
            pltpu.VMEM((R_PER_W,), jnp.int32),
            pltpu.VMEM((2, CHUNK, EMB), jnp.float32),
            pltpu.SemaphoreType.DMA,
        ],
        compiler_params=pltpu.CompilerParams(use_tc_tiling_on_sc=False),
    )
    def gather_kernel(tbl_hbm, gidx_hbm, out_hbm, tbl_v, tbl_s, idx_v, rows_v, sem):
        sid = lax.axis_index("s")
        wid = sid * NC + lax.axis_index("c")
        base = wid * R_PER_W

        @pl.when(sid == 0)
        def _stage_table():
            pltpu.sync_copy(tbl_hbm, tbl_v)
            pltpu.sync_copy(tbl_v, tbl_s)

        plsc.subcore_barrier()
        pltpu.sync_copy(gidx_hbm.at[pl.ds(base, R_PER_W)], idx_v)

        def fire(k, p):
            return [
                pltpu.async_copy(
                    tbl_s.at[
                        idx_v.at[pl.ds((k * STREAMS_PER_CHUNK + j) * IDX_MINOR, IDX_MINOR)]
                    ],
                    rows_v.at[p].at[pl.ds(j * IDX_MINOR, IDX_MINOR)],
                    sem,
                )
                for j in range(STREAMS_PER_CHUNK)
            ]

        handles = fire(0, 0)
        for k in range(NCHUNK):
            nxt = fire(k + 1, (k + 1) % 2) if k + 1 < NCHUNK else []
            for h in handles:
                h.wait()
            handles = nxt
            pltpu.sync_copy(
                rows_v.at[k % 2],
                out_hbm.at[pl.ds(base + k * CHUNK, CHUNK)],
            )

    return gather_kernel(flat_table, gidx_flat)


def _mlp(emb128, W1p, b1, W2, b2, W3, b3):
    BB = 1024
    NSLICE = PADW // 128

    def body(emb_ref, w1_ref, b1_ref, w2_ref, b2_ref, w3_ref, b3_ref, out_ref):
        e = emb_ref[...]
        e3 = e.reshape(BB, NSLICE, 128)
        h = b1_ref[...]
        for g in range(NSLICE):
            eg = e3[:, g, :]
            h = h + jnp.dot(
                eg,
                w1_ref[pl.ds(g * 128, 128), :],
                preferred_element_type=jnp.float32,
            )
        h = jnp.maximum(h, 0.0)
        h = jnp.maximum(
            jnp.dot(h, w2_ref[...], preferred_element_type=jnp.float32)
            + b2_ref[...],
            0.0,
        )
        logits = (
            jnp.dot(h, w3_ref[...], preferred_element_type=jnp.float32)
            + b3_ref[...]
        )
        m = jnp.max(logits, axis=1, keepdims=True)
        ex = jnp.exp(logits - m)
        out_ref[...] = ex / jnp.sum(ex, axis=1, keepdims=True)

    return pl.pallas_call(
        body,
        grid=(B // BB,),
        in_specs=[
            pl.BlockSpec((NSLICE * BB, 128), lambda i: (i, 0)),
            pl.BlockSpec((PADW, HIDDEN), lambda i: (0, 0)),
            pl.BlockSpec((1, HIDDEN), lambda i: (0, 0)),
            pl.BlockSpec((HIDDEN, HIDDEN), lambda i: (0, 0)),
            pl.BlockSpec((1, HIDDEN), lambda i: (0, 0)),
            pl.BlockSpec((HIDDEN, ACTIONS), lambda i: (0, 0)),
            pl.BlockSpec((1, ACTIONS), lambda i: (0, 0)),
        ],
        out_specs=pl.BlockSpec((BB, ACTIONS), lambda i: (i, 0)),
        out_shape=jax.ShapeDtypeStruct((B, ACTIONS), jnp.float32),
    )(emb128, W1p, b1, W2, b2, W3, b3)


def kernel(x, table0, tables, W1, b1, W2, b2, W3, b3):
    flat_table = jnp.concatenate([table0, tables.reshape(-1, EMB)], axis=0)
    offs = jnp.concatenate(
        [
            jnp.zeros((1,), jnp.int32),
            25 + 10 * jnp.arange(NFIELDS - 1, dtype=jnp.int32),
        ]
    )
    core = x.astype(jnp.int32) + offs[None, :]
    gidx = jnp.concatenate(
        [core, jnp.zeros((B, GPB - NFIELDS), jnp.int32)], axis=1
    ).reshape(-1)
    emb_rows = _sc_gather(flat_table, gidx)
    emb128 = emb_rows.reshape(B * PADW // 128, 128)
    W1p = jnp.concatenate([W1, jnp.zeros((PADW - CONCAT, HIDDEN), W1.dtype)])
    return _mlp(
        emb128,
        W1p,
        b1.reshape(1, HIDDEN),
        W2,
        b2.reshape(1, HIDDEN),
        W3,
        b3.reshape(1, ACTIONS),
    )


# speedup vs baseline: 1.2043x; 1.2043x over previous
import functools

import jax
import jax.numpy as jnp
from jax import lax
from jax.experimental import pallas as pl
from jax.experimental.pallas import tpu as pltpu
from jax.experimental.pallas import tpu_sc as plsc

B = 16384
HIDDEN = 256
ACTIONS = 64
EMB = 16
NFIELDS = 17
CONCAT = NFIELDS * EMB
TABLE_ROWS = 25 + (NFIELDS - 1) * 10

GPB = 24
PADW = GPB * EMB
NC, NS = 2, 16
NW = NC * NS
R = B * GPB
R_PER_W = R // NW
IDX_MINOR = 128
B_PER_CHUNK = 128
CHUNK = B_PER_CHUNK * GPB
STREAMS_PER_CHUNK = CHUNK // IDX_MINOR
B_PER_W = B // NW
NCHUNK = B_PER_W // B_PER_CHUNK


def _sc_gather(flat_table, gidx_flat):
    mesh = plsc.VectorSubcoreMesh(core_axis_name="c", subcore_axis_name="s")

    @functools.partial(
        pl.kernel,
        mesh=mesh,
        out_type=jax.ShapeDtypeStruct((R, EMB), jnp.float32),
        scratch_types=[
            pltpu.VMEM((TABLE_ROWS, EMB), jnp.float32),
            pltpu.VMEM_SHARED((TABLE_ROWS, EMB), jnp.float32),
            pltpu.VMEM((R_PER_W,), jnp.int32),
            pltpu.VMEM((2, CHUNK, EMB), jnp.float32),
            pltpu.SemaphoreType.DMA,
        ],
        compiler_params=pltpu.CompilerParams(use_tc_tiling_on_sc=False),
    )
    def gather_kernel(tbl_hbm, gidx_hbm, out_hbm, tbl_v, tbl_s, idx_v, rows_v, sem):
        sid = lax.axis_index("s")
        wid = sid * NC + lax.axis_index("c")
        base = wid * R_PER_W

        @pl.when(sid == 0)
        def _stage_table():
            pltpu.sync_copy(tbl_hbm, tbl_v)
            pltpu.sync_copy(tbl_v, tbl_s)

        plsc.subcore_barrier()
        pltpu.sync_copy(gidx_hbm.at[pl.ds(base, R_PER_W)], idx_v)

        def fire(k, p):
            return [
                pltpu.async_copy(
                    tbl_s.at[
                        idx_v.at[pl.ds((k * STREAMS_PER_CHUNK + j) * IDX_MINOR, IDX_MINOR)]
                    ],
                    rows_v.at[p].at[pl.ds(j * IDX_MINOR, IDX_MINOR)],
                    sem,
                )
                for j in range(STREAMS_PER_CHUNK)
            ]

        handles = fire(0, 0)
        for k in range(NCHUNK):
            nxt = fire(k + 1, (k + 1) % 2) if k + 1 < NCHUNK else []
            for h in handles:
                h.wait()
            handles = nxt
            pltpu.sync_copy(
                rows_v.at[k % 2],
                out_hbm.at[pl.ds(base + k * CHUNK, CHUNK)],
            )

    return gather_kernel(flat_table, gidx_flat)


def _mlp(emb128, W1p, b1, W2, b2, W3, b3):
    BB = 1024
    NSLICE = PADW // 128

    def body(e0_ref, e1_ref, e2_ref, w1_ref, b1_ref, w2_ref, b2_ref, w3_ref, b3_ref, out_ref):
        h = b1_ref[...]
        for g, eref in enumerate((e0_ref, e1_ref, e2_ref)):
            h = h + jnp.dot(
                eref[...],
                w1_ref[pl.ds(g * 128, 128), :],
                preferred_element_type=jnp.float32,
            )
        h = jnp.maximum(h, 0.0)
        h = jnp.maximum(
            jnp.dot(h, w2_ref[...], preferred_element_type=jnp.float32)
            + b2_ref[...],
            0.0,
        )
        logits = (
            jnp.dot(h, w3_ref[...], preferred_element_type=jnp.float32)
            + b3_ref[...]
        )
        m = jnp.max(logits, axis=1, keepdims=True)
        ex = jnp.exp(logits - m)
        out_ref[...] = ex / jnp.sum(ex, axis=1, keepdims=True)

    nb = B // BB
    return pl.pallas_call(
        body,
        grid=(nb,),
        in_specs=[
            pl.BlockSpec((BB, 128), lambda i: (i, 0)),
            pl.BlockSpec((BB, 128), lambda i: (i + nb, 0)),
            pl.BlockSpec((BB, 128), lambda i: (i + 2 * nb, 0)),
            pl.BlockSpec((PADW, HIDDEN), lambda i: (0, 0)),
            pl.BlockSpec((1, HIDDEN), lambda i: (0, 0)),
            pl.BlockSpec((HIDDEN, HIDDEN), lambda i: (0, 0)),
            pl.BlockSpec((1, HIDDEN), lambda i: (0, 0)),
            pl.BlockSpec((HIDDEN, ACTIONS), lambda i: (0, 0)),
            pl.BlockSpec((1, ACTIONS), lambda i: (0, 0)),
        ],
        out_specs=pl.BlockSpec((BB, ACTIONS), lambda i: (i, 0)),
        out_shape=jax.ShapeDtypeStruct((B, ACTIONS), jnp.float32),
    )(emb128, emb128, emb128, W1p, b1, W2, b2, W3, b3)


def kernel(x, table0, tables, W1, b1, W2, b2, W3, b3):
    flat_table = jnp.concatenate([table0, tables.reshape(-1, EMB)], axis=0)
    offs = jnp.concatenate(
        [
            jnp.zeros((1,), jnp.int32),
            25 + 10 * jnp.arange(NFIELDS - 1, dtype=jnp.int32),
        ]
    )
    core = x.astype(jnp.int32) + offs[None, :]
    gidx = jnp.concatenate(
        [
            core[:, 0:8].reshape(-1),
            core[:, 8:16].reshape(-1),
            jnp.concatenate([core[:, 16:17], core[:, 0:7]], axis=1).reshape(-1),
        ]
    )
    emb_rows = _sc_gather(flat_table, gidx)
    emb128 = emb_rows.reshape(B * PADW // 128, 128)
    W1p = jnp.concatenate([W1, jnp.zeros((PADW - CONCAT, HIDDEN), W1.dtype)])
    return _mlp(
        emb128,
        W1p,
        b1.reshape(1, HIDDEN),
        W2,
        b2.reshape(1, HIDDEN),
        W3,
        b3.reshape(1, ACTIONS),
    )


# speedup vs baseline: 1.2575x; 1.0442x over previous
import functools

import jax
import jax.numpy as jnp
from jax import lax
from jax.experimental import pallas as pl
from jax.experimental.pallas import tpu as pltpu
from jax.experimental.pallas import tpu_sc as plsc

B = 16384
HIDDEN = 256
ACTIONS = 64
EMB = 16
NFIELDS = 17
CONCAT = NFIELDS * EMB
TABLE_ROWS = 25 + (NFIELDS - 1) * 10

GPB = 24
PADW = GPB * EMB
NC, NS = 2, 16
NW = NC * NS
R = B * GPB
R_PER_W = R // NW
IDX_MINOR = 128
B_PER_CHUNK = 128
CHUNK = B_PER_CHUNK * GPB
STREAMS_PER_CHUNK = CHUNK // IDX_MINOR
B_PER_W = B // NW
NCHUNK = B_PER_W // B_PER_CHUNK


def _sc_gather(flat_table, gidx_flat):
    mesh = plsc.VectorSubcoreMesh(core_axis_name="c", subcore_axis_name="s")

    @functools.partial(
        pl.kernel,
        mesh=mesh,
        out_type=jax.ShapeDtypeStruct((R, EMB), jnp.float32),
        scratch_types=[
            pltpu.VMEM((TABLE_ROWS, EMB), jnp.float32),
            pltpu.VMEM_SHARED((TABLE_ROWS, EMB), jnp.float32),
            pltpu.VMEM((R_PER_W,), jnp.int32),
            pltpu.VMEM((2, CHUNK, EMB), jnp.float32),
            pltpu.SemaphoreType.DMA,
        ],
        compiler_params=pltpu.CompilerParams(use_tc_tiling_on_sc=False),
    )
    def gather_kernel(tbl_hbm, gidx_hbm, out_hbm, tbl_v, tbl_s, idx_v, rows_v, sem):
        sid = lax.axis_index("s")
        wid = sid * NC + lax.axis_index("c")
        base = wid * R_PER_W

        @pl.when(sid == 0)
        def _stage_table():
            pltpu.sync_copy(tbl_hbm, tbl_v)
            pltpu.sync_copy(tbl_v, tbl_s)

        plsc.subcore_barrier()
        pltpu.sync_copy(gidx_hbm.at[pl.ds(base, R_PER_W)], idx_v)

        def fire(k, p):
            return [
                pltpu.async_copy(
                    tbl_s.at[
                        idx_v.at[pl.ds((k * STREAMS_PER_CHUNK + j) * IDX_MINOR, IDX_MINOR)]
                    ],
                    rows_v.at[p].at[pl.ds(j * IDX_MINOR, IDX_MINOR)],
                    sem,
                )
                for j in range(STREAMS_PER_CHUNK)
            ]

        handles = fire(0, 0)
        for k in range(NCHUNK):
            nxt = fire(k + 1, (k + 1) % 2) if k + 1 < NCHUNK else []
            for h in handles:
                h.wait()
            handles = nxt
            pltpu.sync_copy(
                rows_v.at[k % 2],
                out_hbm.at[pl.ds(base + k * CHUNK, CHUNK)],
            )

    return gather_kernel(flat_table, gidx_flat)


def _mlp(emb128, W1p, b1, W2, b2, W3, b3):
    BB = 1024
    NSLICE = PADW // 128

    def body(e0_ref, e1_ref, e2_ref, w1_ref, b1_ref, w2_ref, b2_ref, w3_ref, b3_ref, out_ref):
        h = b1_ref[...]
        for g, eref in enumerate((e0_ref, e1_ref, e2_ref)):
            h = h + jnp.dot(
                eref[...],
                w1_ref[pl.ds(g * 128, 128), :],
                preferred_element_type=jnp.float32,
            )
        h = jnp.maximum(h, 0.0)
        h = jnp.maximum(
            jnp.dot(h, w2_ref[...], preferred_element_type=jnp.float32)
            + b2_ref[...],
            0.0,
        )
        logits = (
            jnp.dot(h, w3_ref[...], preferred_element_type=jnp.float32)
            + b3_ref[...]
        )
        m = jnp.max(logits, axis=1, keepdims=True)
        ex = jnp.exp(logits - m)
        out_ref[...] = ex / jnp.sum(ex, axis=1, keepdims=True)

    nb = B // BB
    return pl.pallas_call(
        body,
        grid=(nb,),
        in_specs=[
            pl.BlockSpec((BB, 128), lambda i: (i, 0)),
            pl.BlockSpec((BB, 128), lambda i: (i + nb, 0)),
            pl.BlockSpec((BB, 128), lambda i: (i + 2 * nb, 0)),
            pl.BlockSpec((PADW, HIDDEN), lambda i: (0, 0)),
            pl.BlockSpec((1, HIDDEN), lambda i: (0, 0)),
            pl.BlockSpec((HIDDEN, HIDDEN), lambda i: (0, 0)),
            pl.BlockSpec((1, HIDDEN), lambda i: (0, 0)),
            pl.BlockSpec((HIDDEN, ACTIONS), lambda i: (0, 0)),
            pl.BlockSpec((1, ACTIONS), lambda i: (0, 0)),
        ],
        out_specs=pl.BlockSpec((BB, ACTIONS), lambda i: (i, 0)),
        out_shape=jax.ShapeDtypeStruct((B, ACTIONS), jnp.float32),
    )(emb128, emb128, emb128, W1p, b1, W2, b2, W3, b3)


def kernel(x, table0, tables, W1, b1, W2, b2, W3, b3):
    flat_table = jnp.concatenate([table0, tables.reshape(-1, EMB)], axis=0)
    offs = jnp.concatenate(
        [
            jnp.zeros((1,), jnp.int32),
            25 + 10 * jnp.arange(NFIELDS - 1, dtype=jnp.int32),
        ]
    )
    core = x.astype(jnp.int32) + offs[None, :]
    core_pad = jnp.concatenate([core, core[:, :7]], axis=1)
    gidx = core_pad.reshape(B, 3, 8).transpose(1, 0, 2).reshape(-1)
    emb_rows = _sc_gather(flat_table, gidx)
    emb128 = emb_rows.reshape(B * PADW // 128, 128)
    W1p = jnp.concatenate([W1, jnp.zeros((PADW - CONCAT, HIDDEN), W1.dtype)])
    return _mlp(
        emb128,
        W1p,
        b1.reshape(1, HIDDEN),
        W2,
        b2.reshape(1, HIDDEN),
        W3,
        b3.reshape(1, ACTIONS),
    )
